# baseline (speedup 1.0000x reference)
import jax
import jax.numpy as jnp
from jax import lax
from jax.experimental import pallas as pl
from jax.experimental.pallas import tpu as pltpu
from jax.experimental.pallas import tpu_sc as plsc

PAD = 1
B = 4
S = 8192
D = 1024
VROWS = 8194

NC = 2
NS = 16
L = 16

NW = NC * NS
TOK_PER_W = (B * S) // NW
W_PER_ROW = S // TOK_PER_W
CHUNK = 32
NCHUNKS = TOK_PER_W // CHUNK


def _sc_body(ids_hbm, w1, o1, ids_v, stage_v, tot_v, tot_sh, dsem):
    cid = lax.axis_index("c")
    sid = lax.axis_index("s")
    row = 2 * cid + sid // W_PER_ROW
    slot = sid % W_PER_ROW
    tbase = row * S + slot * TOK_PER_W

    pltpu.sync_copy(ids_hbm.at[pl.ds(tbase, TOK_PER_W)], ids_v)

    def cs_body(i, carry):
        v = ids_v[pl.ds(i * L, L)]
        m = jnp.where(v != PAD, 1, 0).astype(jnp.int32)
        c = plsc.cumsum(m) + carry
        ids_v[pl.ds(i * L, L)] = c * m
        return jnp.max(c)

    total = lax.fori_loop(0, TOK_PER_W // L, cs_body, jnp.int32(0))

    stage_v[...] = jnp.full((L,), total, jnp.int32)
    pltpu.sync_copy(stage_v, tot_sh.at[pl.ds(sid * L, L)])
    plsc.subcore_barrier()
    pltpu.sync_copy(tot_sh, tot_v)

    rstart = (sid // W_PER_ROW) * W_PER_ROW
    offset = jnp.int32(0)
    for jj in range(W_PER_ROW):
        j = rstart + jj
        t = jnp.max(tot_v[pl.ds(j * L, L)])
        offset = offset + jnp.where(j < sid, t, 0).astype(jnp.int32)

    def idx_body(i, _):
        e = ids_v[pl.ds(i * L, L)]
        m = jnp.where(e > 0, 1, 0).astype(jnp.int32)
        ids_v[pl.ds(i * L, L)] = e + offset * m + PAD
        return 0

    lax.fori_loop(0, TOK_PER_W // L, idx_body, 0)

    def copy_rows(src_row, dst_tok, nrows):
        pltpu.async_copy(
            w1.at[pl.ds(src_row * D, nrows * D)],
            o1.at[pl.ds(dst_tok * D, nrows * D)], dsem)

    def drain(nrows):
        pltpu.make_async_copy(
            w1.at[pl.ds(0, nrows * D)], o1.at[pl.ds(0, nrows * D)],
            dsem).wait()

    vf = ids_v[pl.ds(0, L)]
    vl = ids_v[pl.ds(TOK_PER_W - L, L)]
    first_w = vf[0]
    last_w = vl[L - 1]
    win_clean = jnp.logical_and(first_w > 1,
                                last_w - first_w == TOK_PER_W - 1)

    @pl.when(win_clean)
    def _():
        copy_rows(first_w, tbase, TOK_PER_W)
        drain(TOK_PER_W)

    @pl.when(jnp.logical_not(win_clean))
    def _():
        def chunk_body(k, _):
            cf = ids_v[pl.ds(k * CHUNK, L)]
            cl = ids_v[pl.ds(k * CHUNK + CHUNK - L, L)]
            first = cf[0]
            last = cl[L - 1]
            clean = jnp.logical_and(first > 1, last - first == CHUNK - 1)

            @pl.when(clean)
            def _():
                copy_rows(first, tbase + k * CHUNK, CHUNK)

            @pl.when(jnp.logical_not(clean))
            def _():
                for g in range(CHUNK // L):
                    v = ids_v[pl.ds(k * CHUNK + g * L, L)]
                    for l in range(L):
                        copy_rows(v[l], tbase + k * CHUNK + g * L + l, 1)

            @pl.when(k >= 1)
            def _():
                drain(CHUNK)

            return 0

        lax.fori_loop(0, NCHUNKS, chunk_body, 0)
        drain(CHUNK)


@jax.jit
def _sc_embed(ids_flat, weights):
    mesh = plsc.VectorSubcoreMesh(
        core_axis_name="c", subcore_axis_name="s",
        num_cores=NC, num_subcores=NS)
    f = pl.kernel(
        _sc_body,
        out_type=jax.ShapeDtypeStruct((B * S * D,), jnp.float32),
        mesh=mesh,
        compiler_params=pltpu.CompilerParams(needs_layout_passes=False),
        scratch_types=[
            pltpu.VMEM((TOK_PER_W,), jnp.int32),
            pltpu.VMEM((L,), jnp.int32),
            pltpu.VMEM((NS * L,), jnp.int32),
            pltpu.VMEM_SHARED((NS * L,), jnp.int32),
            pltpu.SemaphoreType.DMA,
        ],
    )
    return f(ids_flat, weights.reshape(-1))


def kernel(input_ids, weights):
    out = _sc_embed(input_ids.reshape(-1), weights)
    return out.reshape(B, S, D)


# speedup vs baseline: 36.7986x; 36.7986x over previous
import jax
import jax.numpy as jnp
from jax import lax
from jax.experimental import pallas as pl
from jax.experimental.pallas import tpu as pltpu
from jax.experimental.pallas import tpu_sc as plsc

PAD = 1
B = 4
S = 8192
D = 1024

NC = 2
NS = 16
L = 16

NW = NC * NS
TOK_PER_W = (B * S) // NW
W_PER_ROW = S // TOK_PER_W
CHUNK = 32
NBUF = 2
NCHUNKS = TOK_PER_W // CHUNK
NSTEPS = NCHUNKS // NBUF


def _sc_body(ids_hbm, w_hbm, out_hbm, ids_v, stage_v, tot_v,
             rows_v, tot_sh, g0, g1, s0, s1):
    gsem = [g0, g1]
    ssem = [s0, s1]
    cid = lax.axis_index("c")
    sid = lax.axis_index("s")
    row = 2 * cid + sid // W_PER_ROW
    slot = sid % W_PER_ROW
    tbase = row * S + slot * TOK_PER_W

    pltpu.sync_copy(ids_hbm.at[pl.ds(tbase, TOK_PER_W)], ids_v)

    def cs_body(i, carry):
        v = ids_v[pl.ds(i * L, L)]
        m = jnp.where(v != PAD, 1, 0).astype(jnp.int32)
        c = plsc.cumsum(m) + carry
        ids_v[pl.ds(i * L, L)] = c * m
        return jnp.max(c)

    total = lax.fori_loop(0, TOK_PER_W // L, cs_body, jnp.int32(0))

    stage_v[...] = jnp.full((L,), total, jnp.int32)
    pltpu.sync_copy(stage_v, tot_sh.at[pl.ds(sid * L, L)])
    plsc.subcore_barrier()
    pltpu.sync_copy(tot_sh, tot_v)

    rstart = (sid // W_PER_ROW) * W_PER_ROW
    offset = jnp.int32(0)
    for jj in range(W_PER_ROW):
        j = rstart + jj
        t = jnp.max(tot_v[pl.ds(j * L, L)])
        offset = offset + jnp.where(j < sid, t, 0).astype(jnp.int32)

    def idx_body(i, _):
        e = ids_v[pl.ds(i * L, L)]
        m = jnp.where(e > 0, 1, 0).astype(jnp.int32)
        ids_v[pl.ds(i * L, L)] = e + offset * m + PAD
        return 0

    lax.fori_loop(0, TOK_PER_W // L, idx_body, 0)

    def gather_start(k, b):
        pltpu.async_copy(
            w_hbm.at[ids_v.at[pl.ds(k * CHUNK, CHUNK)]], rows_v.at[b],
            gsem[b])

    def gather_wait(b):
        pltpu.make_async_copy(
            w_hbm.at[ids_v.at[pl.ds(0, CHUNK)]], rows_v.at[b],
            gsem[b]).wait()

    def scatter_start(k, b):
        pltpu.async_copy(
            rows_v.at[b], out_hbm.at[pl.ds(tbase + k * CHUNK, CHUNK)],
            ssem[b])

    def scatter_wait(b):
        pltpu.make_async_copy(
            rows_v.at[b], out_hbm.at[pl.ds(0, CHUNK)], ssem[b]).wait()

    for b in range(NBUF):
        gather_start(b, b)

    def pipe_body(step, _):
        for b in range(NBUF):
            k = step * NBUF + b
            gather_wait(b)
            scatter_start(k, b)
            scatter_wait(b)
            gather_start(k + NBUF, b)
        return 0

    lax.fori_loop(0, NSTEPS - 1, pipe_body, 0)

    for b in range(NBUF):
        k = (NSTEPS - 1) * NBUF + b
        gather_wait(b)
        scatter_start(k, b)
        scatter_wait(b)


@jax.jit
def _sc_embed(ids_flat, weights):
    mesh = plsc.VectorSubcoreMesh(
        core_axis_name="c", subcore_axis_name="s",
        num_cores=NC, num_subcores=NS)
    f = pl.kernel(
        _sc_body,
        out_type=jax.ShapeDtypeStruct((B * S, D), jnp.float32),
        mesh=mesh,
        compiler_params=pltpu.CompilerParams(needs_layout_passes=False),
        scratch_types=[
            pltpu.VMEM((TOK_PER_W,), jnp.int32),
            pltpu.VMEM((L,), jnp.int32),
            pltpu.VMEM((NS * L,), jnp.int32),
            pltpu.VMEM((NBUF, CHUNK, D), jnp.float32),
            pltpu.VMEM_SHARED((NS * L,), jnp.int32),
            pltpu.SemaphoreType.DMA,
            pltpu.SemaphoreType.DMA,
            pltpu.SemaphoreType.DMA,
            pltpu.SemaphoreType.DMA,
        ],
    )
    return f(ids_flat, weights)


def kernel(input_ids, weights):
    out = _sc_embed(input_ids.reshape(-1), weights)
    return out.reshape(B, S, D)


# speedup vs baseline: 36.8920x; 1.0025x over previous
import jax
import jax.numpy as jnp
from jax import lax
from jax.experimental import pallas as pl
from jax.experimental.pallas import tpu as pltpu
from jax.experimental.pallas import tpu_sc as plsc

PAD = 1
B = 4
S = 8192
D = 1024

NC = 2
NS = 16
L = 16

NW = NC * NS
TOK_PER_W = (B * S) // NW
W_PER_ROW = S // TOK_PER_W
CHUNK = 16
NBUF = 4
NCHUNKS = TOK_PER_W // CHUNK
NSTEPS = NCHUNKS // NBUF


def _sc_body(ids_hbm, w_hbm, out_hbm, ids_v, stage_v, tot_v,
             rows_v, tot_sh, g0, g1, g2, g3, s0, s1, s2, s3):
    gsem = [g0, g1, g2, g3]
    ssem = [s0, s1, s2, s3]
    cid = lax.axis_index("c")
    sid = lax.axis_index("s")
    row = 2 * cid + sid // W_PER_ROW
    slot = sid % W_PER_ROW
    tbase = row * S + slot * TOK_PER_W

    pltpu.sync_copy(ids_hbm.at[pl.ds(tbase, TOK_PER_W)], ids_v)

    def cs_body(i, carry):
        v = ids_v[pl.ds(i * L, L)]
        m = jnp.where(v != PAD, 1, 0).astype(jnp.int32)
        c = plsc.cumsum(m) + carry
        ids_v[pl.ds(i * L, L)] = c * m
        return jnp.max(c)

    total = lax.fori_loop(0, TOK_PER_W // L, cs_body, jnp.int32(0))

    stage_v[...] = jnp.full((L,), total, jnp.int32)
    pltpu.sync_copy(stage_v, tot_sh.at[pl.ds(sid * L, L)])
    plsc.subcore_barrier()
    pltpu.sync_copy(tot_sh, tot_v)

    rstart = (sid // W_PER_ROW) * W_PER_ROW
    offset = jnp.int32(0)
    for jj in range(W_PER_ROW):
        j = rstart + jj
        t = jnp.max(tot_v[pl.ds(j * L, L)])
        offset = offset + jnp.where(j < sid, t, 0).astype(jnp.int32)

    def idx_body(i, _):
        e = ids_v[pl.ds(i * L, L)]
        m = jnp.where(e > 0, 1, 0).astype(jnp.int32)
        ids_v[pl.ds(i * L, L)] = e + offset * m + PAD
        return 0

    lax.fori_loop(0, TOK_PER_W // L, idx_body, 0)

    def gather_start(k, b):
        pltpu.async_copy(
            w_hbm.at[ids_v.at[pl.ds(k * CHUNK, CHUNK)]], rows_v.at[b],
            gsem[b])

    def gather_wait(b):
        pltpu.make_async_copy(
            w_hbm.at[ids_v.at[pl.ds(0, CHUNK)]], rows_v.at[b],
            gsem[b]).wait()

    def scatter_start(k, b):
        pltpu.async_copy(
            rows_v.at[b], out_hbm.at[pl.ds(tbase + k * CHUNK, CHUNK)],
            ssem[b])

    def scatter_wait(b):
        pltpu.make_async_copy(
            rows_v.at[b], out_hbm.at[pl.ds(0, CHUNK)], ssem[b]).wait()

    for b in range(NBUF):
        gather_start(b, b)

    def pipe_body(step, _):
        for b in range(NBUF):
            k = step * NBUF + b
            gather_wait(b)
            scatter_start(k, b)
            scatter_wait(b)
            gather_start(k + NBUF, b)
        return 0

    lax.fori_loop(0, NSTEPS - 1, pipe_body, 0)

    for b in range(NBUF):
        k = (NSTEPS - 1) * NBUF + b
        gather_wait(b)
        scatter_start(k, b)
        scatter_wait(b)


@jax.jit
def _sc_embed(ids_flat, weights):
    mesh = plsc.VectorSubcoreMesh(
        core_axis_name="c", subcore_axis_name="s",
        num_cores=NC, num_subcores=NS)
    f = pl.kernel(
        _sc_body,
        out_type=jax.ShapeDtypeStruct((B * S, D), jnp.float32),
        mesh=mesh,
        compiler_params=pltpu.CompilerParams(needs_layout_passes=False),
        scratch_types=[
            pltpu.VMEM((TOK_PER_W,), jnp.int32),
            pltpu.VMEM((L,), jnp.int32),
            pltpu.VMEM((NS * L,), jnp.int32),
            pltpu.VMEM((NBUF, CHUNK, D), jnp.float32),
            pltpu.VMEM_SHARED((NS * L,), jnp.int32),
            pltpu.SemaphoreType.DMA,
            pltpu.SemaphoreType.DMA,
            pltpu.SemaphoreType.DMA,
            pltpu.SemaphoreType.DMA,
            pltpu.SemaphoreType.DMA,
            pltpu.SemaphoreType.DMA,
            pltpu.SemaphoreType.DMA,
            pltpu.SemaphoreType.DMA,
        ],
    )
    return f(ids_flat, weights)


def kernel(input_ids, weights):
    out = _sc_embed(input_ids.reshape(-1), weights)
    return out.reshape(B, S, D)


# speedup vs baseline: 39.3504x; 1.0666x over previous
import jax
import jax.numpy as jnp
from jax import lax
from jax.experimental import pallas as pl
from jax.experimental.pallas import tpu as pltpu
from jax.experimental.pallas import tpu_sc as plsc

PAD = 1
B = 4
S = 8192
D = 1024

NC = 2
NS = 16
L = 16

WTOK = S // NS
CHUNK = 16
NPAIRS = WTOK // CHUNK
SLOTS = 2
NSTEPS = NPAIRS // SLOTS


def _sc_body(ids_hbm, w_hbm, out_hbm, idsA_v, idsB_v, stage_v, tot_v,
             rows_v, tot_sh, gA0, gA1, gB0, gB1, sA0, sA1, sB0, sB1):
    gsemA = [gA0, gA1]
    gsemB = [gB0, gB1]
    ssemA = [sA0, sA1]
    ssemB = [sB0, sB1]
    cid = lax.axis_index("c")
    sid = lax.axis_index("s")
    tbaseA = (2 * cid) * S + sid * WTOK
    tbaseB = (2 * cid + 1) * S + sid * WTOK

    pltpu.sync_copy(ids_hbm.at[pl.ds(tbaseA, WTOK)], idsA_v)
    pltpu.sync_copy(ids_hbm.at[pl.ds(tbaseB, WTOK)], idsB_v)

    def make_cs(ref):
        def cs_body(i, carry):
            v = ref[pl.ds(i * L, L)]
            m = jnp.where(v != PAD, 1, 0).astype(jnp.int32)
            c = plsc.cumsum(m) + carry
            ref[pl.ds(i * L, L)] = c * m
            return jnp.max(c)
        return cs_body

    totalA = lax.fori_loop(0, WTOK // L, make_cs(idsA_v), jnp.int32(0))
    totalB = lax.fori_loop(0, WTOK // L, make_cs(idsB_v), jnp.int32(0))

    stage_v[...] = jnp.full((L,), totalA, jnp.int32)
    pltpu.sync_copy(stage_v, tot_sh.at[pl.ds(sid * L, L)])
    stage_v[...] = jnp.full((L,), totalB, jnp.int32)
    pltpu.sync_copy(stage_v, tot_sh.at[pl.ds((NS + sid) * L, L)])
    plsc.subcore_barrier()
    pltpu.sync_copy(tot_sh, tot_v)

    offA = jnp.int32(0)
    offB = jnp.int32(0)
    for j in range(NS):
        tA = jnp.max(tot_v[pl.ds(j * L, L)])
        tB = jnp.max(tot_v[pl.ds((NS + j) * L, L)])
        keep = j < sid
        offA = offA + jnp.where(keep, tA, 0).astype(jnp.int32)
        offB = offB + jnp.where(keep, tB, 0).astype(jnp.int32)

    def make_idx(ref, off):
        def idx_body(i, _):
            e = ref[pl.ds(i * L, L)]
            m = jnp.where(e > 0, 1, 0).astype(jnp.int32)
            ref[pl.ds(i * L, L)] = e + off * m + PAD
            return 0
        return idx_body

    lax.fori_loop(0, WTOK // L, make_idx(idsA_v, offA), 0)
    lax.fori_loop(0, WTOK // L, make_idx(idsB_v, offB), 0)

    def shared_flag(k):
        vA = idsA_v[pl.ds(k * CHUNK, L)]
        vB = idsB_v[pl.ds(k * CHUNK, L)]
        fA, lA = vA[0], vA[L - 1]
        fB, lB = vB[0], vB[L - 1]
        cleanA = jnp.logical_and(fA > 1, lA - fA == CHUNK - 1)
        cleanB = jnp.logical_and(fB > 1, lB - fB == CHUNK - 1)
        return jnp.logical_and(jnp.logical_and(cleanA, cleanB), fA == fB)

    def issue(k, b):
        sh = shared_flag(k)
        pltpu.async_copy(
            w_hbm.at[idsA_v.at[pl.ds(k * CHUNK, CHUNK)]],
            rows_v.at[b, 0], gsemA[b])

        @pl.when(jnp.logical_not(sh))
        def _():
            pltpu.async_copy(
                w_hbm.at[idsB_v.at[pl.ds(k * CHUNK, CHUNK)]],
                rows_v.at[b, 1], gsemB[b])

    def consume(k, b):
        sh = shared_flag(k)
        pltpu.make_async_copy(
            w_hbm.at[idsA_v.at[pl.ds(0, CHUNK)]], rows_v.at[b, 0],
            gsemA[b]).wait()

        @pl.when(jnp.logical_not(sh))
        def _():
            pltpu.make_async_copy(
                w_hbm.at[idsB_v.at[pl.ds(0, CHUNK)]], rows_v.at[b, 1],
                gsemB[b]).wait()

        pltpu.async_copy(
            rows_v.at[b, 0], out_hbm.at[pl.ds(tbaseA + k * CHUNK, CHUNK)],
            ssemA[b])

        @pl.when(sh)
        def _():
            pltpu.async_copy(
                rows_v.at[b, 0],
                out_hbm.at[pl.ds(tbaseB + k * CHUNK, CHUNK)], ssemB[b])

        @pl.when(jnp.logical_not(sh))
        def _():
            pltpu.async_copy(
                rows_v.at[b, 1],
                out_hbm.at[pl.ds(tbaseB + k * CHUNK, CHUNK)], ssemB[b])

        pltpu.make_async_copy(
            rows_v.at[b, 0], out_hbm.at[pl.ds(0, CHUNK)], ssemA[b]).wait()
        pltpu.make_async_copy(
            rows_v.at[b, 1], out_hbm.at[pl.ds(0, CHUNK)], ssemB[b]).wait()

    for b in range(SLOTS):
        issue(b, b)

    def pipe_body(step, _):
        for b in range(SLOTS):
            k = step * SLOTS + b
            consume(k, b)
            issue(k + SLOTS, b)
        return 0

    lax.fori_loop(0, NSTEPS - 1, pipe_body, 0)

    for b in range(SLOTS):
        consume((NSTEPS - 1) * SLOTS + b, b)


@jax.jit
def _sc_embed(ids_flat, weights):
    mesh = plsc.VectorSubcoreMesh(
        core_axis_name="c", subcore_axis_name="s",
        num_cores=NC, num_subcores=NS)
    f = pl.kernel(
        _sc_body,
        out_type=jax.ShapeDtypeStruct((B * S, D), jnp.float32),
        mesh=mesh,
        compiler_params=pltpu.CompilerParams(needs_layout_passes=False),
        scratch_types=[
            pltpu.VMEM((WTOK,), jnp.int32),
            pltpu.VMEM((WTOK,), jnp.int32),
            pltpu.VMEM((L,), jnp.int32),
            pltpu.VMEM((2 * NS * L,), jnp.int32),
            pltpu.VMEM((SLOTS, 2, CHUNK, D), jnp.float32),
            pltpu.VMEM_SHARED((2 * NS * L,), jnp.int32),
            pltpu.SemaphoreType.DMA,
            pltpu.SemaphoreType.DMA,
            pltpu.SemaphoreType.DMA,
            pltpu.SemaphoreType.DMA,
            pltpu.SemaphoreType.DMA,
            pltpu.SemaphoreType.DMA,
            pltpu.SemaphoreType.DMA,
            pltpu.SemaphoreType.DMA,
        ],
    )
    return f(ids_flat, weights)


def kernel(input_ids, weights):
    out = _sc_embed(input_ids.reshape(-1), weights)
    return out.reshape(B, S, D)
